# baseline (speedup 1.0000x reference)
import functools

import jax
import jax.numpy as jnp
from jax import lax
from jax.experimental import pallas as pl
from jax.experimental.pallas import tpu as pltpu
from jax.experimental.pallas import tpu_sc as plsc

N = 10000
E = 320000
D = 128
NC = 2
NS = 16
NW = NC * NS
CHUNK = 128
BLK = 8
NBLK = 10
EPT = CHUNK * BLK * NBLK
TRASH = N
NTRASH = 240
AGG_ROWS = N + NTRASH
ZROWS = 40
NZCHUNK = N // ZROWS
DEG_PAD = 10240
DEG_SUB = 1024


def _sc_agg_body(with_deg, h_hbm, idx_all, z2d, z1d, aggp, degp,
                 idxA, idxB, rows0, rows1, ones_v, zbuf, dzbuf,
                 sem0, sem1, semA, semB, agg_sh, deg_sh):
    c = lax.axis_index("c")
    s = lax.axis_index("s")
    wid = s * NC + c

    pltpu.sync_copy(z2d, zbuf)
    for k in range((NZCHUNK + NS - 1) // NS):
        m = s + NS * k
        @pl.when(m < NZCHUNK)
        def _():
            pltpu.sync_copy(zbuf, agg_sh.at[pl.ds(m * ZROWS, ZROWS)])
    if with_deg:
        @pl.when(s < DEG_PAD // DEG_SUB)
        def _():
            pltpu.sync_copy(z1d, dzbuf)
            pltpu.sync_copy(dzbuf, deg_sh.at[pl.ds(s * DEG_SUB, DEG_SUB)])
        ones16 = jnp.ones((16,), jnp.float32)
        for k in range(8):
            ones_v[pl.ds(k * 16, 16)] = ones16
    plsc.subcore_barrier()

    base = wid * NBLK
    pltpu.sync_copy(idx_all.at[base], idxA)
    pltpu.async_copy(idx_all.at[base + 1], idxB, semB)

    def outer(t, _):
        for ci in range(2 * BLK):
            buf = idxA if ci < BLK else idxB
            i = ci % BLK
            rows_c, sem_c = (rows0, sem0) if ci % 2 == 0 else (rows1, sem1)
            rows_n, sem_n = (rows1, sem1) if ci % 2 == 0 else (rows0, sem0)
            pltpu.async_copy(h_hbm.at[buf.at[i]], rows_c, sem_c).wait()
            if ci == BLK - 1:
                pltpu.make_async_copy(idx_all.at[base], idxB, semB).wait()
            if ci == 2 * BLK - 1:
                pltpu.make_async_copy(idx_all.at[base], idxA, semA).wait()
            pltpu.sync_copy(rows_c, agg_sh.at[buf.at[BLK + i]], add=True)
            if with_deg:
                pltpu.sync_copy(ones_v, deg_sh.at[buf.at[BLK + i]], add=True)
            if ci == BLK - 1:
                nxt = base + lax.rem(2 * t + 2, NBLK)
                pltpu.async_copy(idx_all.at[nxt], idxA, semA)
            if ci == 2 * BLK - 1:
                nxt = base + lax.rem(2 * t + 3, NBLK)
                pltpu.async_copy(idx_all.at[nxt], idxB, semB)
        return 0

    lax.fori_loop(0, NBLK // 2, outer, 0)
    pltpu.make_async_copy(idx_all.at[base + 1], idxB, semB).wait()
    plsc.subcore_barrier()

    for k in range((NZCHUNK + NS - 1) // NS):
        m = s + NS * k
        @pl.when(m < NZCHUNK)
        def _():
            pltpu.sync_copy(agg_sh.at[pl.ds(m * ZROWS, ZROWS)], zbuf)
            pltpu.sync_copy(zbuf, aggp.at[c].at[pl.ds(m * ZROWS, ZROWS)])
    if with_deg:
        @pl.when(s < DEG_PAD // DEG_SUB)
        def _():
            pltpu.sync_copy(deg_sh.at[pl.ds(s * DEG_SUB, DEG_SUB)], dzbuf)
            pltpu.sync_copy(dzbuf,
                            degp.at[pl.ds(c * DEG_PAD + s * DEG_SUB, DEG_SUB)])


def _make_sc_agg(with_deg):
    mesh = plsc.VectorSubcoreMesh(core_axis_name="c", subcore_axis_name="s")
    out_type = (jax.ShapeDtypeStruct((NC, N, D), jnp.float32),
                jax.ShapeDtypeStruct((NC * DEG_PAD,), jnp.float32))
    scratch = [
        pltpu.VMEM((2 * BLK, CHUNK), jnp.int32),
        pltpu.VMEM((2 * BLK, CHUNK), jnp.int32),
        pltpu.VMEM((CHUNK, D), jnp.float32),
        pltpu.VMEM((CHUNK, D), jnp.float32),
        pltpu.VMEM((128,), jnp.float32),
        pltpu.VMEM((ZROWS, D), jnp.float32),
        pltpu.VMEM((DEG_SUB,), jnp.float32),
        pltpu.SemaphoreType.DMA,
        pltpu.SemaphoreType.DMA,
        pltpu.SemaphoreType.DMA,
        pltpu.SemaphoreType.DMA,
        pltpu.VMEM_SHARED((AGG_ROWS, D), jnp.float32),
        pltpu.VMEM_SHARED((DEG_PAD,), jnp.float32),
    ]
    body = functools.partial(_sc_agg_body, with_deg)
    return pl.kernel(body, out_type=out_type, mesh=mesh, scratch_types=scratch,
                     name="sc_agg_deg" if with_deg else "sc_agg")


_sc_agg_with_deg = _make_sc_agg(True)
_sc_agg_plain = _make_sc_agg(False)

TCR = 2000


def _tc_layer1_body(aggA, aggB, degA, degB, x, Wl, Wr, b, h_out, inv_out):
    deg = jnp.maximum(degA[...] + degB[...], 1.0)
    inv = 1.0 / deg
    mean = (aggA[...] + aggB[...]) * inv
    h = jnp.dot(mean, Wl[...], preferred_element_type=jnp.float32)
    h += jnp.dot(x[...], Wr[...], preferred_element_type=jnp.float32)
    h += b[...]
    h_out[...] = jnp.maximum(h, 0.0)
    inv_out[...] = inv


def _tc_layer_body(has_head, aggA, aggB, inv, hp, Wl, Wr, b, *rest):
    mean = (aggA[...] + aggB[...]) * inv[...]
    h = jnp.dot(mean, Wl[...], preferred_element_type=jnp.float32)
    h += jnp.dot(hp[...], Wr[...], preferred_element_type=jnp.float32)
    h += b[...]
    h = jnp.maximum(h, 0.0) + hp[...]
    if has_head:
        Wo, bo, out = rest
        out[...] = jnp.dot(h, Wo[...], preferred_element_type=jnp.float32) + bo[...]
    else:
        (out,) = rest
        out[...] = h


_row_spec = pl.BlockSpec((TCR, D), lambda i: (i, 0))
_col_spec = pl.BlockSpec((TCR, 1), lambda i: (i, 0))
_w_spec = pl.BlockSpec((D, D), lambda i: (0, 0))
_b_spec = pl.BlockSpec((1, D), lambda i: (0, 0))

_tc_layer1 = pl.pallas_call(
    _tc_layer1_body,
    grid=(N // TCR,),
    in_specs=[_row_spec, _row_spec, _col_spec, _col_spec, _row_spec,
              _w_spec, _w_spec, _b_spec],
    out_specs=[_row_spec, _col_spec],
    out_shape=[jax.ShapeDtypeStruct((N, D), jnp.float32),
               jax.ShapeDtypeStruct((N, 1), jnp.float32)],
)

_tc_layer_mid = pl.pallas_call(
    functools.partial(_tc_layer_body, False),
    grid=(N // TCR,),
    in_specs=[_row_spec, _row_spec, _col_spec, _row_spec,
              _w_spec, _w_spec, _b_spec],
    out_specs=_row_spec,
    out_shape=jax.ShapeDtypeStruct((N, D), jnp.float32),
)

_tc_layer_last = pl.pallas_call(
    functools.partial(_tc_layer_body, True),
    grid=(N // TCR,),
    in_specs=[_row_spec, _row_spec, _col_spec, _row_spec,
              _w_spec, _w_spec, _b_spec, _w_spec, _b_spec],
    out_specs=_row_spec,
    out_shape=jax.ShapeDtypeStruct((N, D), jnp.float32),
)


def kernel(x, edge_index, Wl1, bl1, Wr1, g1, be1, Wl2, bl2, Wr2, g2, be2,
           Wl3, bl3, Wr3, g3, be3, Wo, bo):
    src = edge_index[0].reshape(NW, E // NW)
    dst = edge_index[1].reshape(NW, E // NW)
    pad = EPT - E // NW
    src = jnp.concatenate([src, jnp.zeros((NW, pad), jnp.int32)], axis=1)
    trash = TRASH + jnp.tile(jnp.arange(pad, dtype=jnp.int32), (NW, 1)) % NTRASH
    dst = jnp.concatenate([dst, trash], axis=1)
    src_b = src.reshape(NW, NBLK, BLK, CHUNK)
    dst_b = dst.reshape(NW, NBLK, BLK, CHUNK)
    idx_all = jnp.concatenate([src_b, dst_b], axis=2).reshape(
        NW * NBLK, 2 * BLK, CHUNK)
    z2d = jnp.zeros((ZROWS, D), jnp.float32)
    z1d = jnp.zeros((DEG_SUB,), jnp.float32)

    def fold(Wl, bl, Wr, g, be):
        s = (g / jnp.sqrt(1.0 + 1e-5))[None, :]
        return Wl * s, Wr * s, (bl[None, :] * s + be[None, :])

    Wl1f, Wr1f, b1f = fold(Wl1, bl1, Wr1, g1, be1)
    Wl2f, Wr2f, b2f = fold(Wl2, bl2, Wr2, g2, be2)
    Wl3f, Wr3f, b3f = fold(Wl3, bl3, Wr3, g3, be3)

    aggp, degp = _sc_agg_with_deg(x, idx_all, z2d, z1d)
    degA = degp[0:N, None]
    degB = degp[DEG_PAD:DEG_PAD + N, None]
    h1, inv = _tc_layer1(aggp[0], aggp[1], degA, degB, x, Wl1f, Wr1f, b1f)
    aggp2, _ = _sc_agg_plain(h1, idx_all, z2d, z1d)
    h2 = _tc_layer_mid(aggp2[0], aggp2[1], inv, h1, Wl2f, Wr2f, b2f)
    aggp3, _ = _sc_agg_plain(h2, idx_all, z2d, z1d)
    return _tc_layer_last(aggp3[0], aggp3[1], inv, h2, Wl3f, Wr3f, b3f,
                          Wo, bo[None, :])


# speedup vs baseline: 1.0030x; 1.0030x over previous
import functools

import jax
import jax.numpy as jnp
from jax import lax
from jax.experimental import pallas as pl
from jax.experimental.pallas import tpu as pltpu
from jax.experimental.pallas import tpu_sc as plsc

N = 10000
E = 320000
D = 128
NC = 2
NS = 16
NW = NC * NS
CHUNK = 128
BLK = 8
NBLK = 10
EPT = CHUNK * BLK * NBLK
TRASH = N
NTRASH = 240
AGG_ROWS = N + NTRASH
ZROWS = 40
NZCHUNK = N // ZROWS
DEG_PAD = 10240
DEG_SUB = 1024


def _sc_agg_body(with_deg, h_hbm, idx_all, z2d, z1d, aggp, degp,
                 idxA, idxB, rows0, ones_v, zbuf, dzbuf,
                 sem0, agg_sh, deg_sh):
    c = lax.axis_index("c")
    s = lax.axis_index("s")
    wid = s * NC + c

    pltpu.sync_copy(z2d, zbuf)
    for k in range((NZCHUNK + NS - 1) // NS):
        m = s + NS * k
        @pl.when(m < NZCHUNK)
        def _():
            pltpu.sync_copy(zbuf, agg_sh.at[pl.ds(m * ZROWS, ZROWS)])
    if with_deg:
        @pl.when(s < DEG_PAD // DEG_SUB)
        def _():
            pltpu.sync_copy(z1d, dzbuf)
            pltpu.sync_copy(dzbuf, deg_sh.at[pl.ds(s * DEG_SUB, DEG_SUB)])
        ones16 = jnp.ones((16,), jnp.float32)
        for k in range(8):
            ones_v[pl.ds(k * 16, 16)] = ones16
    plsc.subcore_barrier()

    RPT = EPT // CHUNK
    pltpu.sync_copy(idx_all.at[pl.ds(wid * 2 * RPT, RPT)], idxA)
    pltpu.sync_copy(idx_all.at[pl.ds(wid * 2 * RPT + RPT, RPT)], idxB)

    def step(j, _):
        pltpu.async_copy(h_hbm.at[idxA.at[j]], rows0, sem0).wait()
        pltpu.sync_copy(rows0, agg_sh.at[idxB.at[j]], add=True)
        if with_deg:
            pltpu.sync_copy(ones_v, deg_sh.at[idxB.at[j]], add=True)
        return 0

    lax.fori_loop(0, RPT, step, 0)
    plsc.subcore_barrier()

    for k in range((NZCHUNK + NS - 1) // NS):
        m = s + NS * k
        @pl.when(m < NZCHUNK)
        def _():
            pltpu.sync_copy(agg_sh.at[pl.ds(m * ZROWS, ZROWS)], zbuf)
            pltpu.sync_copy(zbuf, aggp.at[c].at[pl.ds(m * ZROWS, ZROWS)])
    if with_deg:
        @pl.when(s < DEG_PAD // DEG_SUB)
        def _():
            pltpu.sync_copy(deg_sh.at[pl.ds(s * DEG_SUB, DEG_SUB)], dzbuf)
            pltpu.sync_copy(dzbuf,
                            degp.at[pl.ds(c * DEG_PAD + s * DEG_SUB, DEG_SUB)])


def _make_sc_agg(with_deg):
    mesh = plsc.VectorSubcoreMesh(core_axis_name="c", subcore_axis_name="s")
    out_type = (jax.ShapeDtypeStruct((NC, N, D), jnp.float32),
                jax.ShapeDtypeStruct((NC * DEG_PAD,), jnp.float32))
    scratch = [
        pltpu.VMEM((EPT // CHUNK, CHUNK), jnp.int32),
        pltpu.VMEM((EPT // CHUNK, CHUNK), jnp.int32),
        pltpu.VMEM((CHUNK, D), jnp.float32),
        pltpu.VMEM((128,), jnp.float32),
        pltpu.VMEM((ZROWS, D), jnp.float32),
        pltpu.VMEM((DEG_SUB,), jnp.float32),
        pltpu.SemaphoreType.DMA,
        pltpu.VMEM_SHARED((AGG_ROWS, D), jnp.float32),
        pltpu.VMEM_SHARED((DEG_PAD,), jnp.float32),
    ]
    body = functools.partial(_sc_agg_body, with_deg)
    return pl.kernel(body, out_type=out_type, mesh=mesh, scratch_types=scratch,
                     name="sc_agg_deg" if with_deg else "sc_agg")


_sc_agg_with_deg = _make_sc_agg(True)
_sc_agg_plain = _make_sc_agg(False)

TCR = 2000


def _tc_layer1_body(aggA, aggB, degA, degB, x, Wl, Wr, b, h_out, inv_out):
    deg = jnp.maximum(degA[...] + degB[...], 1.0)
    inv = 1.0 / deg
    mean = (aggA[...] + aggB[...]) * inv
    h = jnp.dot(mean, Wl[...], preferred_element_type=jnp.float32)
    h += jnp.dot(x[...], Wr[...], preferred_element_type=jnp.float32)
    h += b[...]
    h_out[...] = jnp.maximum(h, 0.0)
    inv_out[...] = inv


def _tc_layer_body(has_head, aggA, aggB, inv, hp, Wl, Wr, b, *rest):
    mean = (aggA[...] + aggB[...]) * inv[...]
    h = jnp.dot(mean, Wl[...], preferred_element_type=jnp.float32)
    h += jnp.dot(hp[...], Wr[...], preferred_element_type=jnp.float32)
    h += b[...]
    h = jnp.maximum(h, 0.0) + hp[...]
    if has_head:
        Wo, bo, out = rest
        out[...] = jnp.dot(h, Wo[...], preferred_element_type=jnp.float32) + bo[...]
    else:
        (out,) = rest
        out[...] = h


_row_spec = pl.BlockSpec((TCR, D), lambda i: (i, 0))
_col_spec = pl.BlockSpec((TCR, 1), lambda i: (i, 0))
_w_spec = pl.BlockSpec((D, D), lambda i: (0, 0))
_b_spec = pl.BlockSpec((1, D), lambda i: (0, 0))

_tc_layer1 = pl.pallas_call(
    _tc_layer1_body,
    grid=(N // TCR,),
    in_specs=[_row_spec, _row_spec, _col_spec, _col_spec, _row_spec,
              _w_spec, _w_spec, _b_spec],
    out_specs=[_row_spec, _col_spec],
    out_shape=[jax.ShapeDtypeStruct((N, D), jnp.float32),
               jax.ShapeDtypeStruct((N, 1), jnp.float32)],
)

_tc_layer_mid = pl.pallas_call(
    functools.partial(_tc_layer_body, False),
    grid=(N // TCR,),
    in_specs=[_row_spec, _row_spec, _col_spec, _row_spec,
              _w_spec, _w_spec, _b_spec],
    out_specs=_row_spec,
    out_shape=jax.ShapeDtypeStruct((N, D), jnp.float32),
)

_tc_layer_last = pl.pallas_call(
    functools.partial(_tc_layer_body, True),
    grid=(N // TCR,),
    in_specs=[_row_spec, _row_spec, _col_spec, _row_spec,
              _w_spec, _w_spec, _b_spec, _w_spec, _b_spec],
    out_specs=_row_spec,
    out_shape=jax.ShapeDtypeStruct((N, D), jnp.float32),
)


def kernel(x, edge_index, Wl1, bl1, Wr1, g1, be1, Wl2, bl2, Wr2, g2, be2,
           Wl3, bl3, Wr3, g3, be3, Wo, bo):
    src = edge_index[0].reshape(NW, E // NW)
    dst = edge_index[1].reshape(NW, E // NW)
    pad = EPT - E // NW
    src = jnp.concatenate([src, jnp.zeros((NW, pad), jnp.int32)], axis=1)
    trash = TRASH + jnp.tile(jnp.arange(pad, dtype=jnp.int32), (NW, 1)) % NTRASH
    dst = jnp.concatenate([dst, trash], axis=1)
    src_b = src.reshape(NW, EPT // CHUNK, CHUNK)
    dst_b = dst.reshape(NW, EPT // CHUNK, CHUNK)
    idx_all = jnp.concatenate([src_b, dst_b], axis=1).reshape(-1, CHUNK)
    z2d = jnp.zeros((ZROWS, D), jnp.float32)
    z1d = jnp.zeros((DEG_SUB,), jnp.float32)

    def fold(Wl, bl, Wr, g, be):
        s = (g / jnp.sqrt(1.0 + 1e-5))[None, :]
        return Wl * s, Wr * s, (bl[None, :] * s + be[None, :])

    Wl1f, Wr1f, b1f = fold(Wl1, bl1, Wr1, g1, be1)
    Wl2f, Wr2f, b2f = fold(Wl2, bl2, Wr2, g2, be2)
    Wl3f, Wr3f, b3f = fold(Wl3, bl3, Wr3, g3, be3)

    aggp, degp = _sc_agg_with_deg(x, idx_all, z2d, z1d)
    degA = degp[0:N, None]
    degB = degp[DEG_PAD:DEG_PAD + N, None]
    h1, inv = _tc_layer1(aggp[0], aggp[1], degA, degB, x, Wl1f, Wr1f, b1f)
    aggp2, _ = _sc_agg_plain(h1, idx_all, z2d, z1d)
    h2 = _tc_layer_mid(aggp2[0], aggp2[1], inv, h1, Wl2f, Wr2f, b2f)
    aggp3, _ = _sc_agg_plain(h2, idx_all, z2d, z1d)
    return _tc_layer_last(aggp3[0], aggp3[1], inv, h2, Wl3f, Wr3f, b3f,
                          Wo, bo[None, :])


# speedup vs baseline: 2.3160x; 2.3090x over previous
import functools

import jax
import jax.numpy as jnp
from jax import lax
from jax.experimental import pallas as pl
from jax.experimental.pallas import tpu as pltpu
from jax.experimental.pallas import tpu_sc as plsc

N = 10000
E = 320000
D = 128
NC = 2
NS = 16
NW = NC * NS
CHUNK = 125
EPT = E // NW
AGG_ROWS = N
ZROWS = 40
NZCHUNK = N // ZROWS
DEG_PAD = 10240
DEG_SUB = 1024


def _sc_agg_body(with_deg, h_hbm, idx_all, z2d, z1d, aggp, degp,
                 idxA, idxB, rows0, ones_v, zbuf, dzbuf,
                 sem0, agg_sh, deg_sh):
    c = lax.axis_index("c")
    s = lax.axis_index("s")
    wid = s * NC + c

    pltpu.sync_copy(z2d, zbuf)
    for k in range((NZCHUNK + NS - 1) // NS):
        m = s + NS * k
        @pl.when(m < NZCHUNK)
        def _():
            pltpu.sync_copy(zbuf, agg_sh.at[pl.ds(m * ZROWS, ZROWS)])
    if with_deg:
        @pl.when(s < DEG_PAD // DEG_SUB)
        def _():
            pltpu.sync_copy(z1d, dzbuf)
            pltpu.sync_copy(dzbuf, deg_sh.at[pl.ds(s * DEG_SUB, DEG_SUB)])
        ones16 = jnp.ones((16,), jnp.float32)
        for k in range(8):
            ones_v[pl.ds(k * 16, 16)] = ones16
    plsc.subcore_barrier()

    RPT = EPT // CHUNK
    pltpu.sync_copy(idx_all.at[pl.ds(wid * 2 * RPT, RPT)], idxA)
    pltpu.sync_copy(idx_all.at[pl.ds(wid * 2 * RPT + RPT, RPT)], idxB)

    def step(j, _):
        pltpu.async_copy(h_hbm.at[idxA.at[j]], rows0, sem0).wait()
        pltpu.sync_copy(rows0, agg_sh.at[idxB.at[j]], add=True)
        if with_deg:
            pltpu.sync_copy(ones_v.at[pl.ds(0, CHUNK)], deg_sh.at[idxB.at[j]],
                            add=True)
        return 0

    lax.fori_loop(0, RPT, step, 0)
    plsc.subcore_barrier()

    for k in range((NZCHUNK + NS - 1) // NS):
        m = s + NS * k
        @pl.when(m < NZCHUNK)
        def _():
            pltpu.sync_copy(agg_sh.at[pl.ds(m * ZROWS, ZROWS)], zbuf)
            pltpu.sync_copy(zbuf, aggp.at[c].at[pl.ds(m * ZROWS, ZROWS)])
    if with_deg:
        @pl.when(s < DEG_PAD // DEG_SUB)
        def _():
            pltpu.sync_copy(deg_sh.at[pl.ds(s * DEG_SUB, DEG_SUB)], dzbuf)
            pltpu.sync_copy(dzbuf,
                            degp.at[pl.ds(c * DEG_PAD + s * DEG_SUB, DEG_SUB)])


def _make_sc_agg(with_deg):
    mesh = plsc.VectorSubcoreMesh(core_axis_name="c", subcore_axis_name="s")
    out_type = (jax.ShapeDtypeStruct((NC, N, D), jnp.float32),
                jax.ShapeDtypeStruct((NC * DEG_PAD,), jnp.float32))
    scratch = [
        pltpu.VMEM((EPT // CHUNK, CHUNK), jnp.int32),
        pltpu.VMEM((EPT // CHUNK, CHUNK), jnp.int32),
        pltpu.VMEM((CHUNK, D), jnp.float32),
        pltpu.VMEM((128,), jnp.float32),
        pltpu.VMEM((ZROWS, D), jnp.float32),
        pltpu.VMEM((DEG_SUB,), jnp.float32),
        pltpu.SemaphoreType.DMA,
        pltpu.VMEM_SHARED((AGG_ROWS, D), jnp.float32),
        pltpu.VMEM_SHARED((DEG_PAD,), jnp.float32),
    ]
    body = functools.partial(_sc_agg_body, with_deg)
    return pl.kernel(body, out_type=out_type, mesh=mesh, scratch_types=scratch,
                     name="sc_agg_deg" if with_deg else "sc_agg")


_sc_agg_with_deg = _make_sc_agg(True)
_sc_agg_plain = _make_sc_agg(False)

TCR = 2000


def _tc_layer1_body(aggA, aggB, degA, degB, x, Wl, Wr, b, h_out, inv_out):
    deg = jnp.maximum(degA[...] + degB[...], 1.0)
    inv = 1.0 / deg
    mean = (aggA[...] + aggB[...]) * inv
    h = jnp.dot(mean, Wl[...], preferred_element_type=jnp.float32)
    h += jnp.dot(x[...], Wr[...], preferred_element_type=jnp.float32)
    h += b[...]
    h_out[...] = jnp.maximum(h, 0.0)
    inv_out[...] = inv


def _tc_layer_body(has_head, aggA, aggB, inv, hp, Wl, Wr, b, *rest):
    mean = (aggA[...] + aggB[...]) * inv[...]
    h = jnp.dot(mean, Wl[...], preferred_element_type=jnp.float32)
    h += jnp.dot(hp[...], Wr[...], preferred_element_type=jnp.float32)
    h += b[...]
    h = jnp.maximum(h, 0.0) + hp[...]
    if has_head:
        Wo, bo, out = rest
        out[...] = jnp.dot(h, Wo[...], preferred_element_type=jnp.float32) + bo[...]
    else:
        (out,) = rest
        out[...] = h


_row_spec = pl.BlockSpec((TCR, D), lambda i: (i, 0))
_col_spec = pl.BlockSpec((TCR, 1), lambda i: (i, 0))
_w_spec = pl.BlockSpec((D, D), lambda i: (0, 0))
_b_spec = pl.BlockSpec((1, D), lambda i: (0, 0))

_tc_layer1 = pl.pallas_call(
    _tc_layer1_body,
    grid=(N // TCR,),
    in_specs=[_row_spec, _row_spec, _col_spec, _col_spec, _row_spec,
              _w_spec, _w_spec, _b_spec],
    out_specs=[_row_spec, _col_spec],
    out_shape=[jax.ShapeDtypeStruct((N, D), jnp.float32),
               jax.ShapeDtypeStruct((N, 1), jnp.float32)],
)

_tc_layer_mid = pl.pallas_call(
    functools.partial(_tc_layer_body, False),
    grid=(N // TCR,),
    in_specs=[_row_spec, _row_spec, _col_spec, _row_spec,
              _w_spec, _w_spec, _b_spec],
    out_specs=_row_spec,
    out_shape=jax.ShapeDtypeStruct((N, D), jnp.float32),
)

_tc_layer_last = pl.pallas_call(
    functools.partial(_tc_layer_body, True),
    grid=(N // TCR,),
    in_specs=[_row_spec, _row_spec, _col_spec, _row_spec,
              _w_spec, _w_spec, _b_spec, _w_spec, _b_spec],
    out_specs=_row_spec,
    out_shape=jax.ShapeDtypeStruct((N, D), jnp.float32),
)


def kernel(x, edge_index, Wl1, bl1, Wr1, g1, be1, Wl2, bl2, Wr2, g2, be2,
           Wl3, bl3, Wr3, g3, be3, Wo, bo):
    src_b = edge_index[0].reshape(NW, EPT // CHUNK, CHUNK)
    dst_b = edge_index[1].reshape(NW, EPT // CHUNK, CHUNK)
    idx_all = jnp.concatenate([src_b, dst_b], axis=1).reshape(-1, CHUNK)
    z2d = jnp.zeros((ZROWS, D), jnp.float32)
    z1d = jnp.zeros((DEG_SUB,), jnp.float32)

    def fold(Wl, bl, Wr, g, be):
        s = (g / jnp.sqrt(1.0 + 1e-5))[None, :]
        return Wl * s, Wr * s, (bl[None, :] * s + be[None, :])

    Wl1f, Wr1f, b1f = fold(Wl1, bl1, Wr1, g1, be1)
    Wl2f, Wr2f, b2f = fold(Wl2, bl2, Wr2, g2, be2)
    Wl3f, Wr3f, b3f = fold(Wl3, bl3, Wr3, g3, be3)

    aggp, degp = _sc_agg_with_deg(x, idx_all, z2d, z1d)
    degA = degp[0:N, None]
    degB = degp[DEG_PAD:DEG_PAD + N, None]
    h1, inv = _tc_layer1(aggp[0], aggp[1], degA, degB, x, Wl1f, Wr1f, b1f)
    aggp2, _ = _sc_agg_plain(h1, idx_all, z2d, z1d)
    h2 = _tc_layer_mid(aggp2[0], aggp2[1], inv, h1, Wl2f, Wr2f, b2f)
    aggp3, _ = _sc_agg_plain(h2, idx_all, z2d, z1d)
    return _tc_layer_last(aggp3[0], aggp3[1], inv, h2, Wl3f, Wr3f, b3f,
                          Wo, bo[None, :])


# speedup vs baseline: 3.0224x; 1.3050x over previous
import functools

import jax
import jax.numpy as jnp
from jax import lax
from jax.experimental import pallas as pl
from jax.experimental.pallas import tpu as pltpu
from jax.experimental.pallas import tpu_sc as plsc

N = 10000
E = 320000
D = 128
NC = 2
NS = 16
NW = NC * NS
CHUNK = 125
EPT = E // NW
BLK = 8
NBLK = EPT // (BLK * CHUNK)
AGG_ROWS = N
ZROWS = 40
NZCHUNK = N // ZROWS
DEG_PAD = 10240
DEG_SUB = 1024


def _sc_agg_body(with_deg, h_hbm, idx_all, z2d, z1d, aggp, degp,
                 idx3, rows0, rows1, ones_v, zbuf, dzbuf,
                 semG, semI, agg_sh, deg_sh):
    c = lax.axis_index("c")
    s = lax.axis_index("s")
    wid = s * NC + c

    pltpu.sync_copy(z2d, zbuf)
    for k in range((NZCHUNK + NS - 1) // NS):
        m = s + NS * k
        @pl.when(m < NZCHUNK)
        def _():
            pltpu.sync_copy(zbuf, agg_sh.at[pl.ds(m * ZROWS, ZROWS)])
    if with_deg:
        @pl.when(s < DEG_PAD // DEG_SUB)
        def _():
            pltpu.sync_copy(z1d, dzbuf)
            pltpu.sync_copy(dzbuf, deg_sh.at[pl.ds(s * DEG_SUB, DEG_SUB)])
        ones16 = jnp.ones((16,), jnp.float32)
        for k in range(8):
            ones_v[pl.ds(k * 16, 16)] = ones16
    plsc.subcore_barrier()

    base = wid * NBLK
    pltpu.sync_copy(idx_all.at[base], idx3.at[0])
    pltpu.async_copy(idx_all.at[base + 1], idx3.at[1], semI)
    pltpu.async_copy(h_hbm.at[idx3.at[0].at[0]], rows0, semG)

    def block(t, _):
        p = lax.rem(t, 2)
        cur = idx3.at[p]
        nxt = idx3.at[1 - p]
        for i in range(BLK):
            rows_c, rows_n = (rows0, rows1) if i % 2 == 0 else (rows1, rows0)
            pltpu.make_async_copy(h_hbm.at[cur.at[i]], rows_c, semG).wait()
            if i == BLK - 1:
                pltpu.make_async_copy(idx_all.at[base], nxt, semI).wait()
                pltpu.async_copy(h_hbm.at[nxt.at[0]], rows_n, semG)
            else:
                pltpu.async_copy(h_hbm.at[cur.at[i + 1]], rows_n, semG)
            pltpu.sync_copy(rows_c, agg_sh.at[cur.at[BLK + i]], add=True)
            if with_deg:
                pltpu.sync_copy(ones_v.at[pl.ds(0, CHUNK)],
                                deg_sh.at[cur.at[BLK + i]], add=True)
        pltpu.async_copy(idx_all.at[base + lax.rem(t + 2, NBLK)], cur, semI)
        return 0

    lax.fori_loop(0, NBLK, block, 0)
    pltpu.make_async_copy(h_hbm.at[idx3.at[0].at[0]], rows0, semG).wait()
    pltpu.make_async_copy(idx_all.at[base], idx3.at[0], semI).wait()
    plsc.subcore_barrier()

    for k in range((NZCHUNK + NS - 1) // NS):
        m = s + NS * k
        @pl.when(m < NZCHUNK)
        def _():
            pltpu.sync_copy(agg_sh.at[pl.ds(m * ZROWS, ZROWS)], zbuf)
            pltpu.sync_copy(zbuf, aggp.at[c].at[pl.ds(m * ZROWS, ZROWS)])
    if with_deg:
        @pl.when(s < DEG_PAD // DEG_SUB)
        def _():
            pltpu.sync_copy(deg_sh.at[pl.ds(s * DEG_SUB, DEG_SUB)], dzbuf)
            pltpu.sync_copy(dzbuf,
                            degp.at[pl.ds(c * DEG_PAD + s * DEG_SUB, DEG_SUB)])


def _make_sc_agg(with_deg):
    mesh = plsc.VectorSubcoreMesh(core_axis_name="c", subcore_axis_name="s")
    out_type = (jax.ShapeDtypeStruct((NC, N, D), jnp.float32),
                jax.ShapeDtypeStruct((NC * DEG_PAD,), jnp.float32))
    scratch = [
        pltpu.VMEM((2, 2 * BLK, CHUNK), jnp.int32),
        pltpu.VMEM((CHUNK, D), jnp.float32),
        pltpu.VMEM((CHUNK, D), jnp.float32),
        pltpu.VMEM((128,), jnp.float32),
        pltpu.VMEM((ZROWS, D), jnp.float32),
        pltpu.VMEM((DEG_SUB,), jnp.float32),
        pltpu.SemaphoreType.DMA,
        pltpu.SemaphoreType.DMA,
        pltpu.VMEM_SHARED((AGG_ROWS, D), jnp.float32),
        pltpu.VMEM_SHARED((DEG_PAD,), jnp.float32),
    ]
    body = functools.partial(_sc_agg_body, with_deg)
    return pl.kernel(body, out_type=out_type, mesh=mesh, scratch_types=scratch,
                     name="sc_agg_deg" if with_deg else "sc_agg")


_sc_agg_with_deg = _make_sc_agg(True)
_sc_agg_plain = _make_sc_agg(False)

TCR = 2000


def _tc_layer1_body(aggA, aggB, degA, degB, x, Wl, Wr, b, h_out, inv_out):
    deg = jnp.maximum(degA[...] + degB[...], 1.0)
    inv = 1.0 / deg
    mean = (aggA[...] + aggB[...]) * inv
    h = jnp.dot(mean, Wl[...], preferred_element_type=jnp.float32)
    h += jnp.dot(x[...], Wr[...], preferred_element_type=jnp.float32)
    h += b[...]
    h_out[...] = jnp.maximum(h, 0.0)
    inv_out[...] = inv


def _tc_layer_body(has_head, aggA, aggB, inv, hp, Wl, Wr, b, *rest):
    mean = (aggA[...] + aggB[...]) * inv[...]
    h = jnp.dot(mean, Wl[...], preferred_element_type=jnp.float32)
    h += jnp.dot(hp[...], Wr[...], preferred_element_type=jnp.float32)
    h += b[...]
    h = jnp.maximum(h, 0.0) + hp[...]
    if has_head:
        Wo, bo, out = rest
        out[...] = jnp.dot(h, Wo[...], preferred_element_type=jnp.float32) + bo[...]
    else:
        (out,) = rest
        out[...] = h


_row_spec = pl.BlockSpec((TCR, D), lambda i: (i, 0))
_col_spec = pl.BlockSpec((TCR, 1), lambda i: (i, 0))
_w_spec = pl.BlockSpec((D, D), lambda i: (0, 0))
_b_spec = pl.BlockSpec((1, D), lambda i: (0, 0))

_tc_layer1 = pl.pallas_call(
    _tc_layer1_body,
    grid=(N // TCR,),
    in_specs=[_row_spec, _row_spec, _col_spec, _col_spec, _row_spec,
              _w_spec, _w_spec, _b_spec],
    out_specs=[_row_spec, _col_spec],
    out_shape=[jax.ShapeDtypeStruct((N, D), jnp.float32),
               jax.ShapeDtypeStruct((N, 1), jnp.float32)],
)

_tc_layer_mid = pl.pallas_call(
    functools.partial(_tc_layer_body, False),
    grid=(N // TCR,),
    in_specs=[_row_spec, _row_spec, _col_spec, _row_spec,
              _w_spec, _w_spec, _b_spec],
    out_specs=_row_spec,
    out_shape=jax.ShapeDtypeStruct((N, D), jnp.float32),
)

_tc_layer_last = pl.pallas_call(
    functools.partial(_tc_layer_body, True),
    grid=(N // TCR,),
    in_specs=[_row_spec, _row_spec, _col_spec, _row_spec,
              _w_spec, _w_spec, _b_spec, _w_spec, _b_spec],
    out_specs=_row_spec,
    out_shape=jax.ShapeDtypeStruct((N, D), jnp.float32),
)


def kernel(x, edge_index, Wl1, bl1, Wr1, g1, be1, Wl2, bl2, Wr2, g2, be2,
           Wl3, bl3, Wr3, g3, be3, Wo, bo):
    src_b = edge_index[0].reshape(NW, NBLK, BLK, CHUNK)
    dst_b = edge_index[1].reshape(NW, NBLK, BLK, CHUNK)
    idx_all = jnp.concatenate([src_b, dst_b], axis=2).reshape(
        NW * NBLK, 2 * BLK, CHUNK)
    z2d = jnp.zeros((ZROWS, D), jnp.float32)
    z1d = jnp.zeros((DEG_SUB,), jnp.float32)

    def fold(Wl, bl, Wr, g, be):
        s = (g / jnp.sqrt(1.0 + 1e-5))[None, :]
        return Wl * s, Wr * s, (bl[None, :] * s + be[None, :])

    Wl1f, Wr1f, b1f = fold(Wl1, bl1, Wr1, g1, be1)
    Wl2f, Wr2f, b2f = fold(Wl2, bl2, Wr2, g2, be2)
    Wl3f, Wr3f, b3f = fold(Wl3, bl3, Wr3, g3, be3)

    aggp, degp = _sc_agg_with_deg(x, idx_all, z2d, z1d)
    degA = degp[0:N, None]
    degB = degp[DEG_PAD:DEG_PAD + N, None]
    h1, inv = _tc_layer1(aggp[0], aggp[1], degA, degB, x, Wl1f, Wr1f, b1f)
    aggp2, _ = _sc_agg_plain(h1, idx_all, z2d, z1d)
    h2 = _tc_layer_mid(aggp2[0], aggp2[1], inv, h1, Wl2f, Wr2f, b2f)
    aggp3, _ = _sc_agg_plain(h2, idx_all, z2d, z1d)
    return _tc_layer_last(aggp3[0], aggp3[1], inv, h2, Wl3f, Wr3f, b3f,
                          Wo, bo[None, :])


# speedup vs baseline: 3.1406x; 1.0391x over previous
import functools

import jax
import jax.numpy as jnp
from jax import lax
from jax.experimental import pallas as pl
from jax.experimental.pallas import tpu as pltpu
from jax.experimental.pallas import tpu_sc as plsc

N = 10000
E = 320000
D = 128
NC = 2
NS = 16
NW = NC * NS
CHUNK = 125
EPT = E // NW
BLK = 8
NBLK = EPT // (BLK * CHUNK)
AGG_ROWS = N
ZROWS = 40
NZCHUNK = N // ZROWS
DEG_PAD = 10240
DEG_SUB = 1024


def _sc_agg_body(with_deg, h_hbm, idx_all, z2d, z1d, aggp, degp,
                 idx3, rows0, rows1, ones_v, zbuf, dzbuf,
                 semG, semI, agg_sh, deg_sh):
    c = lax.axis_index("c")
    s = lax.axis_index("s")
    wid = s * NC + c

    pltpu.sync_copy(z2d, zbuf)
    for k in range((NZCHUNK + NS - 1) // NS):
        m = s + NS * k
        @pl.when(m < NZCHUNK)
        def _():
            pltpu.sync_copy(zbuf, agg_sh.at[pl.ds(m * ZROWS, ZROWS)])
    if with_deg:
        @pl.when(s < DEG_PAD // DEG_SUB)
        def _():
            pltpu.sync_copy(z1d, dzbuf)
            pltpu.sync_copy(dzbuf, deg_sh.at[pl.ds(s * DEG_SUB, DEG_SUB)])
        ones16 = jnp.ones((16,), jnp.float32)
        for k in range(8):
            ones_v[pl.ds(k * 16, 16)] = ones16
    plsc.subcore_barrier()

    base = wid * NBLK
    pltpu.sync_copy(idx_all.at[base], idx3.at[0])
    pltpu.async_copy(idx_all.at[base + 1], idx3.at[1], semI)
    pltpu.async_copy(h_hbm.at[idx3.at[0].at[0]], rows0, semG)

    def block(t, _):
        p = lax.rem(t, 2)
        cur = idx3.at[p]
        nxt = idx3.at[1 - p]
        for i in range(BLK):
            rows_c, rows_n = (rows0, rows1) if i % 2 == 0 else (rows1, rows0)
            pltpu.make_async_copy(h_hbm.at[cur.at[i]], rows_c, semG).wait()
            if i == BLK - 1:
                pltpu.make_async_copy(idx_all.at[base], nxt, semI).wait()
                pltpu.async_copy(h_hbm.at[nxt.at[0]], rows_n, semG)
            else:
                pltpu.async_copy(h_hbm.at[cur.at[i + 1]], rows_n, semG)
            pltpu.sync_copy(rows_c, agg_sh.at[cur.at[BLK + i]], add=True)
            if with_deg:
                pltpu.sync_copy(ones_v.at[pl.ds(0, CHUNK)],
                                deg_sh.at[cur.at[BLK + i]], add=True)
        pltpu.async_copy(idx_all.at[base + lax.rem(t + 2, NBLK)], cur, semI)
        return 0

    lax.fori_loop(0, NBLK, block, 0)
    pltpu.make_async_copy(h_hbm.at[idx3.at[0].at[0]], rows0, semG).wait()
    pltpu.make_async_copy(idx_all.at[base], idx3.at[0], semI).wait()
    plsc.subcore_barrier()

    for k in range((NZCHUNK + NS - 1) // NS):
        m = s + NS * k
        @pl.when(m < NZCHUNK)
        def _():
            pltpu.sync_copy(agg_sh.at[pl.ds(m * ZROWS, ZROWS)], zbuf)
            pltpu.sync_copy(zbuf, aggp.at[c].at[pl.ds(m * ZROWS, ZROWS)])
    if with_deg:
        @pl.when(s < DEG_PAD // DEG_SUB)
        def _():
            pltpu.sync_copy(deg_sh.at[pl.ds(s * DEG_SUB, DEG_SUB)], dzbuf)
            pltpu.sync_copy(dzbuf,
                            degp.at[pl.ds(c * DEG_PAD + s * DEG_SUB, DEG_SUB)])


def _make_sc_agg(with_deg):
    mesh = plsc.VectorSubcoreMesh(core_axis_name="c", subcore_axis_name="s")
    out_type = (jax.ShapeDtypeStruct((NC, N, D), jnp.float32),
                jax.ShapeDtypeStruct((NC * DEG_PAD,), jnp.float32))
    scratch = [
        pltpu.VMEM((2, 2 * BLK, CHUNK), jnp.int32),
        pltpu.VMEM((CHUNK, D), jnp.float32),
        pltpu.VMEM((CHUNK, D), jnp.float32),
        pltpu.VMEM((128,), jnp.float32),
        pltpu.VMEM((ZROWS, D), jnp.float32),
        pltpu.VMEM((DEG_SUB,), jnp.float32),
        pltpu.SemaphoreType.DMA,
        pltpu.SemaphoreType.DMA,
        pltpu.VMEM_SHARED((AGG_ROWS, D), jnp.float32),
        pltpu.VMEM_SHARED((DEG_PAD,), jnp.float32),
    ]
    body = functools.partial(_sc_agg_body, with_deg)
    return pl.kernel(body, out_type=out_type, mesh=mesh, scratch_types=scratch,
                     name="sc_agg_deg" if with_deg else "sc_agg")


_sc_agg_with_deg = _make_sc_agg(True)
_sc_agg_plain = _make_sc_agg(False)

TCR = 2000


def _tc_layer1_body(aggp, degp, x, Wl, Wr, b, h_out, inv_out):
    deg = jnp.maximum(degp[0] + degp[1], 1.0)
    inv = 1.0 / deg
    mean = (aggp[0] + aggp[1]) * inv
    h = jnp.dot(mean, Wl[...], preferred_element_type=jnp.float32)
    h += jnp.dot(x[...], Wr[...], preferred_element_type=jnp.float32)
    h += b[...]
    h_out[...] = jnp.maximum(h, 0.0)
    inv_out[...] = inv


def _tc_layer_body(has_head, aggp, inv, hp, Wl, Wr, b, *rest):
    mean = (aggp[0] + aggp[1]) * inv[...]
    h = jnp.dot(mean, Wl[...], preferred_element_type=jnp.float32)
    h += jnp.dot(hp[...], Wr[...], preferred_element_type=jnp.float32)
    h += b[...]
    h = jnp.maximum(h, 0.0) + hp[...]
    if has_head:
        Wo, bo, out = rest
        out[...] = jnp.dot(h, Wo[...], preferred_element_type=jnp.float32) + bo[...]
    else:
        (out,) = rest
        out[...] = h


_row_spec = pl.BlockSpec((TCR, D), lambda i: (i, 0))
_agg_spec = pl.BlockSpec((NC, TCR, D), lambda i: (0, i, 0))
_deg_spec = pl.BlockSpec((NC, TCR, 1), lambda i: (0, i, 0))
_col_spec = pl.BlockSpec((TCR, 1), lambda i: (i, 0))
_w_spec = pl.BlockSpec((D, D), lambda i: (0, 0))
_b_spec = pl.BlockSpec((1, D), lambda i: (0, 0))

_tc_layer1 = pl.pallas_call(
    _tc_layer1_body,
    grid=(N // TCR,),
    in_specs=[_agg_spec, _deg_spec, _row_spec, _w_spec, _w_spec, _b_spec],
    out_specs=[_row_spec, _col_spec],
    out_shape=[jax.ShapeDtypeStruct((N, D), jnp.float32),
               jax.ShapeDtypeStruct((N, 1), jnp.float32)],
)

_tc_layer_mid = pl.pallas_call(
    functools.partial(_tc_layer_body, False),
    grid=(N // TCR,),
    in_specs=[_agg_spec, _col_spec, _row_spec, _w_spec, _w_spec, _b_spec],
    out_specs=_row_spec,
    out_shape=jax.ShapeDtypeStruct((N, D), jnp.float32),
)

_tc_layer_last = pl.pallas_call(
    functools.partial(_tc_layer_body, True),
    grid=(N // TCR,),
    in_specs=[_agg_spec, _col_spec, _row_spec, _w_spec, _w_spec, _b_spec,
              _w_spec, _b_spec],
    out_specs=_row_spec,
    out_shape=jax.ShapeDtypeStruct((N, D), jnp.float32),
)


def kernel(x, edge_index, Wl1, bl1, Wr1, g1, be1, Wl2, bl2, Wr2, g2, be2,
           Wl3, bl3, Wr3, g3, be3, Wo, bo):
    src_b = edge_index[0].reshape(NW, NBLK, BLK, CHUNK)
    dst_b = edge_index[1].reshape(NW, NBLK, BLK, CHUNK)
    idx_all = jnp.concatenate([src_b, dst_b], axis=2).reshape(
        NW * NBLK, 2 * BLK, CHUNK)
    z2d = jnp.zeros((ZROWS, D), jnp.float32)
    z1d = jnp.zeros((DEG_SUB,), jnp.float32)

    def fold(Wl, bl, Wr, g, be):
        s = (g / jnp.sqrt(1.0 + 1e-5))[None, :]
        return Wl * s, Wr * s, (bl[None, :] * s + be[None, :])

    Wl1f, Wr1f, b1f = fold(Wl1, bl1, Wr1, g1, be1)
    Wl2f, Wr2f, b2f = fold(Wl2, bl2, Wr2, g2, be2)
    Wl3f, Wr3f, b3f = fold(Wl3, bl3, Wr3, g3, be3)

    aggp, degp = _sc_agg_with_deg(x, idx_all, z2d, z1d)
    h1, inv = _tc_layer1(aggp, degp.reshape(NC, DEG_PAD, 1), x,
                         Wl1f, Wr1f, b1f)
    aggp2, _ = _sc_agg_plain(h1, idx_all, z2d, z1d)
    h2 = _tc_layer_mid(aggp2, inv, h1, Wl2f, Wr2f, b2f)
    aggp3, _ = _sc_agg_plain(h2, idx_all, z2d, z1d)
    return _tc_layer_last(aggp3, inv, h2, Wl3f, Wr3f, b3f, Wo, bo[None, :])


# speedup vs baseline: 3.1675x; 1.0086x over previous
import functools

import jax
import jax.numpy as jnp
from jax import lax
from jax.experimental import pallas as pl
from jax.experimental.pallas import tpu as pltpu
from jax.experimental.pallas import tpu_sc as plsc

N = 10000
E = 320000
D = 128
NC = 2
NS = 16
NW = NC * NS
CHUNK = 125
EPT = E // NW
BLK = 8
NBLK = EPT // (BLK * CHUNK)
AGG_ROWS = N
ZROWS = 80
NZCHUNK = N // ZROWS
DEG_PAD = 10240
DEG_SUB = 1024


def _sc_agg_body(with_deg, h_hbm, idx_all, z2d, z1d, aggp, degp,
                 idx3, rows0, rows1, ones_v, zbuf, dzbuf,
                 semG, semI, agg_sh, deg_sh):
    c = lax.axis_index("c")
    s = lax.axis_index("s")
    wid = s * NC + c

    pltpu.sync_copy(z2d, zbuf)
    for k in range((NZCHUNK + NS - 1) // NS):
        m = s + NS * k
        @pl.when(m < NZCHUNK)
        def _():
            pltpu.sync_copy(zbuf, agg_sh.at[pl.ds(m * ZROWS, ZROWS)])
    if with_deg:
        @pl.when(s < DEG_PAD // DEG_SUB)
        def _():
            pltpu.sync_copy(z1d, dzbuf)
            pltpu.sync_copy(dzbuf, deg_sh.at[pl.ds(s * DEG_SUB, DEG_SUB)])
        ones16 = jnp.ones((16,), jnp.float32)
        for k in range(8):
            ones_v[pl.ds(k * 16, 16)] = ones16
    plsc.subcore_barrier()

    base = wid * NBLK
    pltpu.sync_copy(idx_all.at[base], idx3.at[0])
    pltpu.async_copy(idx_all.at[base + 1], idx3.at[1], semI)
    pltpu.async_copy(h_hbm.at[idx3.at[0].at[0]], rows0, semG)

    def block(t, _):
        p = lax.rem(t, 2)
        cur = idx3.at[p]
        nxt = idx3.at[1 - p]
        for i in range(BLK):
            rows_c, rows_n = (rows0, rows1) if i % 2 == 0 else (rows1, rows0)
            pltpu.make_async_copy(h_hbm.at[cur.at[i]], rows_c, semG).wait()
            if i == BLK - 1:
                pltpu.make_async_copy(idx_all.at[base], nxt, semI).wait()
                pltpu.async_copy(h_hbm.at[nxt.at[0]], rows_n, semG)
            else:
                pltpu.async_copy(h_hbm.at[cur.at[i + 1]], rows_n, semG)
            pltpu.sync_copy(rows_c, agg_sh.at[cur.at[BLK + i]], add=True)
            if with_deg:
                pltpu.sync_copy(ones_v.at[pl.ds(0, CHUNK)],
                                deg_sh.at[cur.at[BLK + i]], add=True)
        pltpu.async_copy(idx_all.at[base + lax.rem(t + 2, NBLK)], cur, semI)
        return 0

    lax.fori_loop(0, NBLK, block, 0)
    pltpu.make_async_copy(h_hbm.at[idx3.at[0].at[0]], rows0, semG).wait()
    pltpu.make_async_copy(idx_all.at[base], idx3.at[0], semI).wait()
    plsc.subcore_barrier()

    for k in range((NZCHUNK + NS - 1) // NS):
        m = s + NS * k
        @pl.when(m < NZCHUNK)
        def _():
            pltpu.sync_copy(agg_sh.at[pl.ds(m * ZROWS, ZROWS)], zbuf)
            pltpu.sync_copy(zbuf, aggp.at[c].at[pl.ds(m * ZROWS, ZROWS)])
    if with_deg:
        @pl.when(s < DEG_PAD // DEG_SUB)
        def _():
            pltpu.sync_copy(deg_sh.at[pl.ds(s * DEG_SUB, DEG_SUB)], dzbuf)
            pltpu.sync_copy(dzbuf,
                            degp.at[pl.ds(c * DEG_PAD + s * DEG_SUB, DEG_SUB)])


def _make_sc_agg(with_deg):
    mesh = plsc.VectorSubcoreMesh(core_axis_name="c", subcore_axis_name="s")
    out_type = (jax.ShapeDtypeStruct((NC, N, D), jnp.float32),
                jax.ShapeDtypeStruct((NC * DEG_PAD,), jnp.float32))
    scratch = [
        pltpu.VMEM((2, 2 * BLK, CHUNK), jnp.int32),
        pltpu.VMEM((CHUNK, D), jnp.float32),
        pltpu.VMEM((CHUNK, D), jnp.float32),
        pltpu.VMEM((128,), jnp.float32),
        pltpu.VMEM((ZROWS, D), jnp.float32),
        pltpu.VMEM((DEG_SUB,), jnp.float32),
        pltpu.SemaphoreType.DMA,
        pltpu.SemaphoreType.DMA,
        pltpu.VMEM_SHARED((AGG_ROWS, D), jnp.float32),
        pltpu.VMEM_SHARED((DEG_PAD,), jnp.float32),
    ]
    body = functools.partial(_sc_agg_body, with_deg)
    return pl.kernel(body, out_type=out_type, mesh=mesh, scratch_types=scratch,
                     name="sc_agg_deg" if with_deg else "sc_agg")


_sc_agg_with_deg = _make_sc_agg(True)
_sc_agg_plain = _make_sc_agg(False)

TCR = 2000


def _tc_layer1_body(aggp, degp, x, Wl, Wr, b, h_out, inv_out):
    deg = jnp.maximum(degp[0] + degp[1], 1.0)
    inv = 1.0 / deg
    mean = (aggp[0] + aggp[1]) * inv
    h = jnp.dot(mean, Wl[...], preferred_element_type=jnp.float32)
    h += jnp.dot(x[...], Wr[...], preferred_element_type=jnp.float32)
    h += b[...]
    h_out[...] = jnp.maximum(h, 0.0)
    inv_out[...] = inv


def _tc_layer_body(has_head, aggp, inv, hp, Wl, Wr, b, *rest):
    mean = (aggp[0] + aggp[1]) * inv[...]
    h = jnp.dot(mean, Wl[...], preferred_element_type=jnp.float32)
    h += jnp.dot(hp[...], Wr[...], preferred_element_type=jnp.float32)
    h += b[...]
    h = jnp.maximum(h, 0.0) + hp[...]
    if has_head:
        Wo, bo, out = rest
        out[...] = jnp.dot(h, Wo[...], preferred_element_type=jnp.float32) + bo[...]
    else:
        (out,) = rest
        out[...] = h


_row_spec = pl.BlockSpec((TCR, D), lambda i: (i, 0))
_agg_spec = pl.BlockSpec((NC, TCR, D), lambda i: (0, i, 0))
_deg_spec = pl.BlockSpec((NC, TCR, 1), lambda i: (0, i, 0))
_col_spec = pl.BlockSpec((TCR, 1), lambda i: (i, 0))
_w_spec = pl.BlockSpec((D, D), lambda i: (0, 0))
_b_spec = pl.BlockSpec((1, D), lambda i: (0, 0))

_tc_layer1 = pl.pallas_call(
    _tc_layer1_body,
    grid=(N // TCR,),
    in_specs=[_agg_spec, _deg_spec, _row_spec, _w_spec, _w_spec, _b_spec],
    out_specs=[_row_spec, _col_spec],
    out_shape=[jax.ShapeDtypeStruct((N, D), jnp.float32),
               jax.ShapeDtypeStruct((N, 1), jnp.float32)],
)

_tc_layer_mid = pl.pallas_call(
    functools.partial(_tc_layer_body, False),
    grid=(N // TCR,),
    in_specs=[_agg_spec, _col_spec, _row_spec, _w_spec, _w_spec, _b_spec],
    out_specs=_row_spec,
    out_shape=jax.ShapeDtypeStruct((N, D), jnp.float32),
)

_tc_layer_last = pl.pallas_call(
    functools.partial(_tc_layer_body, True),
    grid=(N // TCR,),
    in_specs=[_agg_spec, _col_spec, _row_spec, _w_spec, _w_spec, _b_spec,
              _w_spec, _b_spec],
    out_specs=_row_spec,
    out_shape=jax.ShapeDtypeStruct((N, D), jnp.float32),
)


def kernel(x, edge_index, Wl1, bl1, Wr1, g1, be1, Wl2, bl2, Wr2, g2, be2,
           Wl3, bl3, Wr3, g3, be3, Wo, bo):
    src_b = edge_index[0].reshape(NW, NBLK, BLK, CHUNK)
    dst_b = edge_index[1].reshape(NW, NBLK, BLK, CHUNK)
    idx_all = jnp.concatenate([src_b, dst_b], axis=2).reshape(
        NW * NBLK, 2 * BLK, CHUNK)
    z2d = jnp.zeros((ZROWS, D), jnp.float32)
    z1d = jnp.zeros((DEG_SUB,), jnp.float32)

    def fold(Wl, bl, Wr, g, be):
        s = (g / jnp.sqrt(1.0 + 1e-5))[None, :]
        return Wl * s, Wr * s, (bl[None, :] * s + be[None, :])

    Wl1f, Wr1f, b1f = fold(Wl1, bl1, Wr1, g1, be1)
    Wl2f, Wr2f, b2f = fold(Wl2, bl2, Wr2, g2, be2)
    Wl3f, Wr3f, b3f = fold(Wl3, bl3, Wr3, g3, be3)

    aggp, degp = _sc_agg_with_deg(x, idx_all, z2d, z1d)
    h1, inv = _tc_layer1(aggp, degp.reshape(NC, DEG_PAD, 1), x,
                         Wl1f, Wr1f, b1f)
    aggp2, _ = _sc_agg_plain(h1, idx_all, z2d, z1d)
    h2 = _tc_layer_mid(aggp2, inv, h1, Wl2f, Wr2f, b2f)
    aggp3, _ = _sc_agg_plain(h2, idx_all, z2d, z1d)
    return _tc_layer_last(aggp3, inv, h2, Wl3f, Wr3f, b3f, Wo, bo[None, :])
